# baseline (speedup 1.0000x reference)
import functools

import jax
import jax.numpy as jnp
from jax import lax
from jax.experimental import pallas as pl
from jax.experimental.pallas import tpu as pltpu
from jax.experimental.pallas import tpu_sc as plsc

NC, NS = 2, 16
NW = NC * NS
BB = 128


def _make_kernel(batch, seq, D):
    n_tc = batch // BB
    n_tr = D // 8
    b_per_w = BB * seq
    mesh = plsc.VectorSubcoreMesh(
        core_axis_name="c", subcore_axis_name="s", num_cores=NC, num_subcores=NS
    )

    @functools.partial(
        pl.kernel,
        out_type=jax.ShapeDtypeStruct((seq, n_tr, n_tc, 8, BB), jnp.float32),
        mesh=mesh,
        scratch_types=[
            pltpu.VMEM((b_per_w,), jnp.int32),
            pltpu.VMEM((seq, BB), jnp.int32),
            pltpu.VMEM((4, BB, D), jnp.float32),
            pltpu.VMEM((2, n_tr, 8, BB + 1), jnp.float32),
            pltpu.SemaphoreType.DMA,
            pltpu.SemaphoreType.DMA,
        ],
        compiler_params=pltpu.CompilerParams(
            use_tc_tiling_on_sc=False, needs_layout_passes=False
        ),
    )
    def emb_kernel(idx_hbm, table_hbm, out_hbm, idx_v, idsT, gbuf, tbuf, gsem, ssem):
        wid = lax.axis_index("s") * NC + lax.axis_index("c")
        base = wid * b_per_w
        pltpu.sync_copy(idx_hbm.at[pl.ds(base, b_per_w)], idx_v)

        lane = lax.iota(jnp.int32, 16)
        biota = lane * seq
        pvecs = [lane + 16 * p for p in range(D // 16)]
        trvecs = [lax.shift_right_logical(pvecs[p], 3) for p in range(D // 16)]
        jrvecs = [lax.bitwise_and(pvecs[p], 7) for p in range(D // 16)]

        @plsc.parallel_loop(0, seq, step=1, unroll=2)
        def _ids_body(s):
            for k in range(BB // 16):
                v = plsc.load_gather(idx_v, [biota + (16 * k * seq + s)])
                idsT[s, pl.ds(16 * k, 16)] = v

        def start_gather(s):
            pltpu.async_copy(table_hbm.at[idsT.at[s]], gbuf.at[s % 4], gsem)

        def wait_gather_one():
            pltpu.make_async_copy(
                table_hbm.at[pl.ds(0, BB)], gbuf.at[0], gsem
            ).wait()

        def wait_store_unit():
            pltpu.make_async_copy(
                tbuf.at[0, :, :, pl.ds(0, BB)], out_hbm.at[0, :, 0], ssem
            ).wait()

        def transpose_block(s, h):
            g = s % 4
            @plsc.parallel_loop(0, BB, step=1, unroll=8)
            def _t_body(t):
                tvec = jnp.full((16,), 0, jnp.int32) + t
                for p in range(D // 16):
                    v = gbuf[g, t, pl.ds(16 * p, 16)]
                    plsc.store_scatter(tbuf.at[h], [trvecs[p], jrvecs[p], tvec], v)

        def start_store(s, h):
            pltpu.async_copy(
                tbuf.at[h, :, :, pl.ds(0, BB)],
                out_hbm.at[s, :, wid],
                ssem,
            )

        start_gather(0)
        start_gather(1)
        start_gather(2)

        def step(s, h):
            wait_gather_one()

            @pl.when(s < seq - 3)
            def _():
                start_gather(s + 3)

            transpose_block(s, h)

            @pl.when(s >= 2)
            def _():
                wait_store_unit()

            start_store(s, h)

        def main_body(i, carry):
            step(2 * i, 0)
            step(2 * i + 1, 1)
            return carry

        lax.fori_loop(0, seq // 2, main_body, 0)

        for _u in range(2):
            wait_store_unit()

    return emb_kernel


def _make_depad(V, D):
    BI = 128
    n_blocks = V // BI
    tail_i = V - n_blocks * BI
    tail_w = n_blocks % NW
    mesh = plsc.VectorSubcoreMesh(
        core_axis_name="c", subcore_axis_name="s", num_cores=NC, num_subcores=NS
    )

    @functools.partial(
        pl.kernel,
        out_type=jax.ShapeDtypeStruct((V // 2, 2 * D), jnp.float32),
        mesh=mesh,
        scratch_types=[
            pltpu.VMEM((2, D, BI + 1), jnp.float32),
            pltpu.VMEM((2, BI // 2, 2 * D), jnp.float32),
            pltpu.SemaphoreType.DMA,
            pltpu.SemaphoreType.DMA,
        ],
        compiler_params=pltpu.CompilerParams(
            use_tc_tiling_on_sc=True, needs_layout_passes=False
        ),
    )
    def depad_kernel(tableT_hbm, out_hbm, bin_v, bout_v, rsem, wsem):
        wid = lax.axis_index("s") * NC + lax.axis_index("c")
        my_n = (n_blocks - wid + NW - 1) // NW

        lane = lax.iota(jnp.int32, 16)
        jvecs = [lane + 16 * p for p in range(D // 16)]

        def read_block(i, hb):
            kb = wid + NW * i
            for tr in range(D // 8):
                pltpu.async_copy(
                    tableT_hbm.at[pl.ds(8 * tr, 8), pl.ds(BI * kb, BI)],
                    bin_v.at[hb, pl.ds(8 * tr, 8), pl.ds(0, BI)],
                    rsem,
                )

        def wait_read():
            for tr in range(D // 8):
                pltpu.make_async_copy(
                    tableT_hbm.at[pl.ds(0, 8), pl.ds(0, BI)],
                    bin_v.at[0, pl.ds(0, 8), pl.ds(0, BI)],
                    rsem,
                ).wait()

        def wait_write():
            pltpu.make_async_copy(
                bout_v.at[0], out_hbm.at[pl.ds(0, BI // 2)], wsem
            ).wait()

        def transpose(hb, width):
            @plsc.parallel_loop(0, width, step=1, unroll=8)
            def _i_body(i):
                row = lax.shift_right_logical(i, 1)
                half = lax.bitwise_and(i, 1) * D
                ivec = jnp.full((16,), 0, jnp.int32) + i
                for p in range(D // 16):
                    v = plsc.load_gather(bin_v.at[hb], [jvecs[p], ivec])
                    bout_v[hb, row, pl.ds(half + 16 * p, 16)] = v

        def write_block(i, hb):
            kb = wid + NW * i
            pltpu.async_copy(
                bout_v.at[hb], out_hbm.at[pl.ds(BI // 2 * kb, BI // 2)], wsem
            )

        @pl.when(my_n > 0)
        def _():
            read_block(0, 0)

            def body(i, carry):
                hb = i % 2

                @pl.when(i + 1 < my_n)
                def _():
                    read_block(i + 1, (i + 1) % 2)

                wait_read()

                @pl.when(i >= 2)
                def _():
                    wait_write()

                transpose(hb, BI)
                write_block(i, hb)
                return carry

            lax.fori_loop(0, my_n, body, 0)
            wait_write()

            @pl.when(my_n > 1)
            def _():
                wait_write()

    del tail_i, tail_w
    return depad_kernel, n_blocks * BI


@jax.jit
def kernel(input_ids, weight):
    batch, seq = input_ids.shape
    V, D = weight.shape
    B = batch * seq
    idx = input_ids.reshape(B).astype(jnp.int32)
    depad, v_cov = _make_depad(V, D)
    w2 = depad(weight.T)
    if v_cov < V:
        tail = weight[v_cov:].reshape((V - v_cov) // 2, 2 * D)
        w2 = lax.dynamic_update_slice(w2, tail, (v_cov // 2, 0))
    wlin = w2.reshape(V, D)
    out5 = _make_kernel(batch, seq, D)(idx, wlin)
    return out5.transpose(2, 4, 0, 1, 3).reshape(batch, seq, D)


# speedup vs baseline: 1.2612x; 1.2612x over previous
import functools

import jax
import jax.numpy as jnp
from jax import lax
from jax.experimental import pallas as pl
from jax.experimental.pallas import tpu as pltpu
from jax.experimental.pallas import tpu_sc as plsc

NC, NS = 2, 16
NW = NC * NS
BB = 128


def _make_kernel(batch, seq, D):
    n_tc = batch // BB
    n_tr = D // 8
    b_per_w = BB * seq
    mesh = plsc.VectorSubcoreMesh(
        core_axis_name="c", subcore_axis_name="s", num_cores=NC, num_subcores=NS
    )

    @functools.partial(
        pl.kernel,
        out_type=jax.ShapeDtypeStruct((seq, n_tr, n_tc, 8, BB), jnp.float32),
        mesh=mesh,
        scratch_types=[
            pltpu.VMEM((b_per_w,), jnp.int32),
            pltpu.VMEM((seq, BB), jnp.int32),
            pltpu.VMEM((4, BB, D), jnp.float32),
            pltpu.VMEM((2, n_tr, 8, BB + 1), jnp.float32),
            pltpu.SemaphoreType.DMA,
            pltpu.SemaphoreType.DMA,
        ],
        compiler_params=pltpu.CompilerParams(
            use_tc_tiling_on_sc=False, needs_layout_passes=False
        ),
    )
    def emb_kernel(idx_hbm, table_hbm, out_hbm, idx_v, idsT, gbuf, tbuf, gsem, ssem):
        wid = lax.axis_index("s") * NC + lax.axis_index("c")
        base = wid * b_per_w
        pltpu.sync_copy(idx_hbm.at[pl.ds(base, b_per_w)], idx_v)

        lane = lax.iota(jnp.int32, 16)
        biota = lane * seq
        pvecs = [lane + 16 * p for p in range(D // 16)]
        trvecs = [lax.shift_right_logical(pvecs[p], 3) for p in range(D // 16)]
        jrvecs = [lax.bitwise_and(pvecs[p], 7) for p in range(D // 16)]

        @plsc.parallel_loop(0, seq, step=1, unroll=2)
        def _ids_body(s):
            for k in range(BB // 16):
                v = plsc.load_gather(idx_v, [biota + (16 * k * seq + s)])
                idsT[s, pl.ds(16 * k, 16)] = v

        def start_gather(s):
            pltpu.async_copy(table_hbm.at[idsT.at[s]], gbuf.at[s % 4], gsem)

        def wait_gather_one():
            pltpu.make_async_copy(
                table_hbm.at[pl.ds(0, BB)], gbuf.at[0], gsem
            ).wait()

        def wait_store_unit():
            pltpu.make_async_copy(
                tbuf.at[0, :, :, pl.ds(0, BB)], out_hbm.at[0, :, 0], ssem
            ).wait()

        def transpose_block(s, h):
            g = s % 4
            @plsc.parallel_loop(0, BB, step=1, unroll=8)
            def _t_body(t):
                tvec = jnp.full((16,), 0, jnp.int32) + t
                for p in range(D // 16):
                    v = gbuf[g, t, pl.ds(16 * p, 16)]
                    plsc.store_scatter(tbuf.at[h], [trvecs[p], jrvecs[p], tvec], v)

        def start_store(s, h):
            pltpu.async_copy(
                tbuf.at[h, :, :, pl.ds(0, BB)],
                out_hbm.at[s, :, wid],
                ssem,
            )

        start_gather(0)
        start_gather(1)
        start_gather(2)

        def step(s, h):
            wait_gather_one()

            @pl.when(s < seq - 3)
            def _():
                start_gather(s + 3)

            transpose_block(s, h)

            @pl.when(s >= 2)
            def _():
                wait_store_unit()

            start_store(s, h)

        def main_body(i, carry):
            step(2 * i, 0)
            step(2 * i + 1, 1)
            return carry

        lax.fori_loop(0, seq // 2, main_body, 0)

        for _u in range(2):
            wait_store_unit()

    return emb_kernel


def _make_depad(V, D):
    CR = 256
    n_chunks = V // CR
    mesh = plsc.VectorSubcoreMesh(
        core_axis_name="c", subcore_axis_name="s", num_cores=NC, num_subcores=NS
    )

    @functools.partial(
        pl.kernel,
        out_type=jax.ShapeDtypeStruct((V // 2, 2 * D), jnp.float32),
        mesh=mesh,
        scratch_types=[
            pltpu.VMEM((2, CR, D), jnp.float32),
            pltpu.VMEM((2, CR // 2, 2 * D), jnp.float32),
            pltpu.SemaphoreType.DMA,
            pltpu.SemaphoreType.DMA,
        ],
        compiler_params=pltpu.CompilerParams(use_tc_tiling_on_sc=True),
    )
    def depad_kernel(table_hbm, out_hbm, bin_v, bout_v, rsem, wsem):
        wid = lax.axis_index("s") * NC + lax.axis_index("c")
        my_n = (n_chunks - wid + NW - 1) // NW

        def read_chunk(i, hb):
            c = wid + NW * i
            pltpu.async_copy(table_hbm.at[pl.ds(CR * c, CR)], bin_v.at[hb], rsem)

        def wait_read():
            pltpu.make_async_copy(
                table_hbm.at[pl.ds(0, CR)], bin_v.at[0], rsem
            ).wait()

        def wait_write():
            pltpu.make_async_copy(
                bout_v.at[0], out_hbm.at[pl.ds(0, CR // 2)], wsem
            ).wait()

        def compact(hb):
            @plsc.parallel_loop(0, CR, step=1, unroll=8)
            def _r_body(r):
                half = lax.bitwise_and(r, 1) * D
                for p in range(D // 16):
                    v = bin_v[hb, r, pl.ds(16 * p, 16)]
                    bout_v[hb, lax.shift_right_logical(r, 1),
                           pl.ds(half + 16 * p, 16)] = v

        def write_chunk(i, hb):
            c = wid + NW * i
            pltpu.async_copy(
                bout_v.at[hb], out_hbm.at[pl.ds(CR // 2 * c, CR // 2)], wsem
            )

        @pl.when(my_n > 0)
        def _():
            read_chunk(0, 0)

            def body(i, carry):
                hb = i % 2

                @pl.when(i + 1 < my_n)
                def _():
                    read_chunk(i + 1, (i + 1) % 2)

                wait_read()

                @pl.when(i >= 2)
                def _():
                    wait_write()

                compact(hb)
                write_chunk(i, hb)
                return carry

            lax.fori_loop(0, my_n, body, 0)
            wait_write()

            @pl.when(my_n > 1)
            def _():
                wait_write()

    return depad_kernel


@jax.jit
def kernel(input_ids, weight):
    batch, seq = input_ids.shape
    V, D = weight.shape
    B = batch * seq
    idx = input_ids.reshape(B).astype(jnp.int32)
    w2 = _make_depad(V, D)(weight)
    v_cov = (V // 256) * 256
    if v_cov < V:
        tail = weight[v_cov:].reshape((V - v_cov) // 2, 2 * D)
        w2 = lax.dynamic_update_slice(w2, tail, (v_cov // 2, 0))
    wlin = w2.reshape(V, D)
    out5 = _make_kernel(batch, seq, D)(idx, wlin)
    return out5.transpose(2, 4, 0, 1, 3).reshape(batch, seq, D)


# speedup vs baseline: 1.2659x; 1.0037x over previous
import functools

import jax
import jax.numpy as jnp
from jax import lax
from jax.experimental import pallas as pl
from jax.experimental.pallas import tpu as pltpu
from jax.experimental.pallas import tpu_sc as plsc

NC, NS = 2, 16
NW = NC * NS
BB = 128


def _make_kernel(batch, seq, D):
    n_tc = batch // BB
    n_tr = D // 8
    b_per_w = BB * seq
    mesh = plsc.VectorSubcoreMesh(
        core_axis_name="c", subcore_axis_name="s", num_cores=NC, num_subcores=NS
    )

    @functools.partial(
        pl.kernel,
        out_type=jax.ShapeDtypeStruct((seq, n_tr, n_tc, 8, BB), jnp.float32),
        mesh=mesh,
        scratch_types=[
            pltpu.VMEM((b_per_w,), jnp.int32),
            pltpu.VMEM((seq, BB), jnp.int32),
            pltpu.VMEM((4, BB, D), jnp.float32),
            pltpu.VMEM((2, n_tr, 8, BB + 1), jnp.float32),
            pltpu.SemaphoreType.DMA,
            pltpu.SemaphoreType.DMA,
        ],
        compiler_params=pltpu.CompilerParams(
            use_tc_tiling_on_sc=False, needs_layout_passes=False
        ),
    )
    def emb_kernel(idx_hbm, table_hbm, out_hbm, idx_v, idsT, gbuf, tbuf, gsem, ssem):
        wid = lax.axis_index("s") * NC + lax.axis_index("c")
        base = wid * b_per_w
        pltpu.sync_copy(idx_hbm.at[pl.ds(base, b_per_w)], idx_v)

        lane = lax.iota(jnp.int32, 16)
        biota = lane * seq
        pvecs = [lane + 16 * p for p in range(D // 16)]
        trvecs = [lax.shift_right_logical(pvecs[p], 3) for p in range(D // 16)]
        jrvecs = [lax.bitwise_and(pvecs[p], 7) for p in range(D // 16)]

        @plsc.parallel_loop(0, seq, step=1, unroll=2)
        def _ids_body(s):
            for k in range(BB // 16):
                v = plsc.load_gather(idx_v, [biota + (16 * k * seq + s)])
                idsT[s, pl.ds(16 * k, 16)] = v

        def start_gather(s):
            pltpu.async_copy(table_hbm.at[idsT.at[s]], gbuf.at[s % 4], gsem)

        def wait_gather_one():
            pltpu.make_async_copy(
                table_hbm.at[pl.ds(0, BB)], gbuf.at[0], gsem
            ).wait()

        def wait_store_unit():
            pltpu.make_async_copy(
                tbuf.at[0, :, :, pl.ds(0, BB)], out_hbm.at[0, :, 0], ssem
            ).wait()

        def transpose_block(s, h):
            g = s % 4
            @plsc.parallel_loop(0, BB, step=1, unroll=8)
            def _t_body(t):
                tvec = jnp.full((16,), 0, jnp.int32) + t
                for p in range(D // 16):
                    v = gbuf[g, t, pl.ds(16 * p, 16)]
                    plsc.store_scatter(tbuf.at[h], [trvecs[p], jrvecs[p], tvec], v)

        def start_store(s, h):
            pltpu.async_copy(
                tbuf.at[h, :, :, pl.ds(0, BB)],
                out_hbm.at[s, :, wid],
                ssem,
            )

        start_gather(0)
        start_gather(1)
        start_gather(2)

        def step(s, h):
            wait_gather_one()

            @pl.when(s < seq - 3)
            def _():
                start_gather(s + 3)

            transpose_block(s, h)

            @pl.when(s >= 2)
            def _():
                wait_store_unit()

            start_store(s, h)

        def main_body(i, carry):
            step(2 * i, 0)
            step(2 * i + 1, 1)
            return carry

        lax.fori_loop(0, seq // 2, main_body, 0)

        for _u in range(2):
            wait_store_unit()

    return emb_kernel


@jax.jit
def kernel(input_ids, weight):
    batch, seq = input_ids.shape
    V, D = weight.shape
    B = batch * seq
    idx = input_ids.reshape(B).astype(jnp.int32)
    out5 = _make_kernel(batch, seq, D)(idx, weight)
    return out5.transpose(2, 4, 0, 1, 3).reshape(batch, seq, D)
